# baseline (speedup 1.0000x reference)
import functools

import jax
import jax.numpy as jnp
from jax import lax
from jax.experimental import pallas as pl
from jax.experimental.pallas import tpu as pltpu
from jax.experimental.pallas import tpu_sc as plsc

N = 10000
D = 128
NC = 2
NS = 16
NW = NC * NS
NB = 80
BATCH = 128
EPW = NB * BATCH
EP = NW * EPW
NP = 10240
RPT = NP // NS
BLK = 256
GRID = NP // BLK


def _sc_mesh():
    return plsc.VectorSubcoreMesh(
        core_axis_name="c", subcore_axis_name="s",
        num_cores=NC, num_subcores=NS)


def _deg_body(dst_hbm, out0, out1, dst_v, zbuf, ones_v, acc):
    c = lax.axis_index("c")
    s = lax.axis_index("s")
    wid = c * jnp.int32(NS) + s

    def fill_z(i, carry):
        zbuf[pl.ds(i * jnp.int32(16), 16)] = jnp.zeros((16,), jnp.float32)
        return carry

    lax.fori_loop(jnp.int32(0), jnp.int32(RPT // 16), fill_z, 0)

    def fill_o(i, carry):
        ones_v[pl.ds(i * jnp.int32(16), 16)] = jnp.ones((16,), jnp.float32)
        return carry

    lax.fori_loop(jnp.int32(0), jnp.int32(BATCH // 16), fill_o, 0)

    rows = pl.ds(s * jnp.int32(RPT), RPT)
    pltpu.sync_copy(dst_hbm.at[pl.ds(wid * jnp.int32(NB), NB)], dst_v)
    pltpu.sync_copy(zbuf, acc.at[rows])
    plsc.subcore_barrier()

    def body(j, carry):
        pltpu.sync_copy(ones_v, acc.at[dst_v.at[j]], add=True)
        return carry

    lax.fori_loop(jnp.int32(0), jnp.int32(NB), body, 0)
    plsc.subcore_barrier()

    @pl.when(c == 0)
    def _():
        pltpu.sync_copy(acc.at[rows], out0.at[rows])

    @pl.when(c == 1)
    def _():
        pltpu.sync_copy(acc.at[rows], out1.at[rows])


_deg_call = functools.partial(
    pl.kernel,
    out_type=(
        jax.ShapeDtypeStruct((NP,), jnp.float32),
        jax.ShapeDtypeStruct((NP,), jnp.float32),
    ),
    mesh=_sc_mesh(),
    scratch_types=[
        pltpu.VMEM((NB, BATCH), jnp.int32),
        pltpu.VMEM((RPT,), jnp.float32),
        pltpu.VMEM((BATCH,), jnp.float32),
        pltpu.VMEM_SHARED((NP,), jnp.float32),
    ],
)(_deg_body)


CH = 16
TB = EP // BATCH
NBC0 = TB // NS


def _emit_chunks(h_hbm, src_hbm, dst_hbm, acc, src_v, dst_v, buf,
                 gsem, isem, base, nchunk):
    i32 = jnp.int32
    for k in range(nchunk):
        pk = i32(k % 2)
        pn = i32((k + 1) % 2)
        if k + 1 < nchunk:
            ia = pltpu.async_copy(
                src_hbm.at[pl.ds(base + i32((k + 1) * CH), CH)],
                src_v.at[pn], isem)
            ib = pltpu.async_copy(
                dst_hbm.at[pl.ds(base + i32((k + 1) * CH), CH)],
                dst_v.at[pn], isem)

        def inner(jj, carry, k=k, pk=pk):
            j = i32(k * CH) + jj
            p = lax.rem(j, i32(2))
            pnx = lax.rem(j + i32(1), i32(2))

            @pl.when(jj < i32(CH - 1))
            def _():
                pltpu.async_copy(h_hbm.at[src_v.at[pk, jj + i32(1)]],
                                 buf.at[pnx], gsem.at[pnx])

            pltpu.make_async_copy(h_hbm.at[src_v.at[pk, jj]],
                                  buf.at[p], gsem.at[p]).wait()
            pltpu.sync_copy(buf.at[p], acc.at[dst_v.at[pk, jj]], add=True)
            return carry

        lax.fori_loop(i32(0), i32(CH), inner, 0)
        if k + 1 < nchunk:
            ia.wait()
            ib.wait()
            j0 = (k + 1) * CH
            pltpu.async_copy(h_hbm.at[src_v.at[pn, i32(0)]],
                             buf.at[i32(j0 % 2)], gsem.at[i32(j0 % 2)])


def _agg_body(h_hbm, src_hbm, dst_hbm, out0,
              src_v, dst_v, buf, acc, gsem, isem):
    c = lax.axis_index("c")
    s = lax.axis_index("s")
    i32 = jnp.int32

    @pl.when(c == 0)
    def _():
        base = s * i32(NBC0)
        pltpu.sync_copy(src_hbm.at[pl.ds(base, CH)], src_v.at[i32(0)])
        pltpu.sync_copy(dst_hbm.at[pl.ds(base, CH)], dst_v.at[i32(0)])
        rows = pl.ds(s * i32(RPT), RPT)
        pltpu.sync_copy(h_hbm.at[rows], acc.at[rows])
        pltpu.async_copy(h_hbm.at[src_v.at[i32(0), i32(0)]],
                         buf.at[i32(0)], gsem.at[i32(0)])
        plsc.subcore_barrier()
        _emit_chunks(h_hbm, src_hbm, dst_hbm, acc, src_v, dst_v, buf,
                     gsem, isem, base, NBC0 // CH)
        plsc.subcore_barrier()
        pltpu.sync_copy(acc.at[rows], out0.at[rows])


_agg_call = functools.partial(
    pl.kernel,
    out_type=jax.ShapeDtypeStruct((NP, D), jnp.float32),
    mesh=_sc_mesh(),
    scratch_types=[
        pltpu.VMEM((2, CH, BATCH), jnp.int32),
        pltpu.VMEM((2, CH, BATCH), jnp.int32),
        pltpu.VMEM((2, BATCH, D), jnp.float32),
        pltpu.VMEM_SHARED((NP, D), jnp.float32),
        pltpu.SemaphoreType.DMA((2,)),
        pltpu.SemaphoreType.DMA,
    ],
)(_agg_body)


def _tc1_body(deg0_ref, deg1_ref, x_ref, w_ref, h_ref, dinv_ref):
    d = deg0_ref[0, 0, :] + deg1_ref[0, 0, :] + 1.0
    di = lax.rsqrt(d)
    h = jnp.dot(x_ref[...], w_ref[...], preferred_element_type=jnp.float32,
                precision=lax.Precision.HIGHEST)
    h_ref[...] = di[:, None] * h
    dinv_ref[0, 0, :] = di


def _tc2_body(s0_ref, dinv_ref, b_ref, out_ref):
    di = dinv_ref[0, 0, :][:, None]
    h = jnp.maximum(di * s0_ref[...] + b_ref[...][None, :], 0.0)
    out_ref[...] = di * h


def _tc3_body(s0_ref, dinv_ref, w_ref, b_ref, out_ref):
    di = dinv_ref[0, 0, :][:, None]
    a = di * s0_ref[...]
    out_ref[...] = (
        jnp.dot(a, w_ref[...], preferred_element_type=jnp.float32,
                precision=lax.Precision.HIGHEST)
        + b_ref[...][None, :]
    )


def _row_spec(width):
    return pl.BlockSpec((BLK, width), lambda i: (i, 0))


def _vec_spec():
    return pl.BlockSpec((1, 1, BLK), lambda i: (i, 0, 0))


def _full_spec(r, c):
    return pl.BlockSpec((r, c), lambda i: (0, 0))


_tc1_call = pl.pallas_call(
    _tc1_body,
    grid=(GRID,),
    in_specs=[_vec_spec(), _vec_spec(), _row_spec(D), _full_spec(D, D)],
    out_specs=[_row_spec(D), _vec_spec()],
    out_shape=[
        jax.ShapeDtypeStruct((NP, D), jnp.float32),
        jax.ShapeDtypeStruct((GRID, 1, BLK), jnp.float32),
    ],
)

_tc2_call = pl.pallas_call(
    _tc2_body,
    grid=(GRID,),
    in_specs=[
        _row_spec(D),
        _vec_spec(),
        pl.BlockSpec((D,), lambda i: (0,)),
    ],
    out_specs=_row_spec(D),
    out_shape=jax.ShapeDtypeStruct((NP, D), jnp.float32),
)

_tc3_call = pl.pallas_call(
    _tc3_body,
    grid=(GRID,),
    in_specs=[
        _row_spec(D),
        _vec_spec(),
        _full_spec(D, D),
        pl.BlockSpec((D,), lambda i: (0,)),
    ],
    out_specs=_row_spec(D),
    out_shape=jax.ShapeDtypeStruct((NP, D), jnp.float32),
)


@jax.jit
def _run(x, src, dst, W1, b1, W_cat, b_cat):
    pad = EP - src.shape[0]
    src_p = jnp.concatenate(
        [src, jnp.zeros((pad,), jnp.int32)]).reshape(TB, BATCH)
    dst_p = jnp.concatenate(
        [dst, N + jnp.arange(pad, dtype=jnp.int32) % (NP - N)],
    ).reshape(TB, BATCH)
    xp = jnp.zeros((NP, D), jnp.float32).at[:N].set(x)

    deg0, deg1 = _deg_call(dst_p)
    hp, dinv = _tc1_call(
        deg0.reshape(GRID, 1, BLK), deg1.reshape(GRID, 1, BLK), xp, W1)
    s1 = _agg_call(hp, src_p, dst_p)
    hp2 = _tc2_call(s1, dinv, b1)
    s2 = _agg_call(hp2, src_p, dst_p)
    out = _tc3_call(s2, dinv, W_cat, b_cat)
    return out[:N, :64], out[:N, 64:]


def kernel(x, edge_index, W1, b1, W_mu, b_mu, W_var, b_var):
    with jax.enable_x64(False):
        src = edge_index[0].astype(jnp.int32)
        dst = edge_index[1].astype(jnp.int32)
        W_cat = jnp.concatenate([W_mu, W_var], axis=1)
        b_cat = jnp.concatenate([b_mu, b_var], axis=0)
        mu, lv = _run(x.astype(jnp.float32), src, dst,
                      W1.astype(jnp.float32), b1.astype(jnp.float32),
                      W_cat.astype(jnp.float32), b_cat.astype(jnp.float32))
    return mu.astype(jnp.float64), lv.astype(jnp.float64)


# speedup vs baseline: 1.0008x; 1.0008x over previous
import functools

import jax
import jax.numpy as jnp
from jax import lax
from jax.experimental import pallas as pl
from jax.experimental.pallas import tpu as pltpu
from jax.experimental.pallas import tpu_sc as plsc

N = 10000
D = 128
NC = 2
NS = 16
NW = NC * NS
NB = 80
BATCH = 128
EPW = NB * BATCH
EP = NW * EPW
NP = 10240
RPT = NP // NS
BLK = 256
GRID = NP // BLK


def _sc_mesh():
    return plsc.VectorSubcoreMesh(
        core_axis_name="c", subcore_axis_name="s",
        num_cores=NC, num_subcores=NS)


def _deg_body(dst_hbm, out0, out1, dst_v, zbuf, ones_v, acc):
    c = lax.axis_index("c")
    s = lax.axis_index("s")
    wid = c * jnp.int32(NS) + s

    def fill_z(i, carry):
        zbuf[pl.ds(i * jnp.int32(16), 16)] = jnp.zeros((16,), jnp.float32)
        return carry

    lax.fori_loop(jnp.int32(0), jnp.int32(RPT // 16), fill_z, 0)

    def fill_o(i, carry):
        ones_v[pl.ds(i * jnp.int32(16), 16)] = jnp.ones((16,), jnp.float32)
        return carry

    lax.fori_loop(jnp.int32(0), jnp.int32(BATCH // 16), fill_o, 0)

    rows = pl.ds(s * jnp.int32(RPT), RPT)
    pltpu.sync_copy(dst_hbm.at[pl.ds(wid * jnp.int32(NB), NB)], dst_v)
    pltpu.sync_copy(zbuf, acc.at[rows])
    plsc.subcore_barrier()

    def body(j, carry):
        pltpu.sync_copy(ones_v, acc.at[dst_v.at[j]], add=True)
        return carry

    lax.fori_loop(jnp.int32(0), jnp.int32(NB), body, 0)
    plsc.subcore_barrier()

    @pl.when(c == 0)
    def _():
        pltpu.sync_copy(acc.at[rows], out0.at[rows])

    @pl.when(c == 1)
    def _():
        pltpu.sync_copy(acc.at[rows], out1.at[rows])


_deg_call = functools.partial(
    pl.kernel,
    out_type=(
        jax.ShapeDtypeStruct((NP,), jnp.float32),
        jax.ShapeDtypeStruct((NP,), jnp.float32),
    ),
    mesh=_sc_mesh(),
    scratch_types=[
        pltpu.VMEM((NB, BATCH), jnp.int32),
        pltpu.VMEM((RPT,), jnp.float32),
        pltpu.VMEM((BATCH,), jnp.float32),
        pltpu.VMEM_SHARED((NP,), jnp.float32),
    ],
)(_deg_body)


CH = 16
TB = EP // BATCH
NBC0 = TB // NS


def _emit_chunks(h_hbm, src_hbm, dst_hbm, acc, src_v, dst_v, buf,
                 gsem, isem, base, nchunk):
    i32 = jnp.int32

    def chunk_body(k, carry):
        pk = lax.rem(k, i32(2))
        pn = lax.rem(k + i32(1), i32(2))
        more = k + i32(1) < i32(nchunk)

        @pl.when(more)
        def _():
            pltpu.async_copy(
                src_hbm.at[pl.ds(base + (k + i32(1)) * i32(CH), CH)],
                src_v.at[pn], isem)
            pltpu.async_copy(
                dst_hbm.at[pl.ds(base + (k + i32(1)) * i32(CH), CH)],
                dst_v.at[pn], isem)

        def inner(jj, icarry):
            p = lax.rem(jj, i32(2))
            pnx = lax.rem(jj + i32(1), i32(2))

            @pl.when(jj < i32(CH - 1))
            def _():
                pltpu.async_copy(h_hbm.at[src_v.at[pk, jj + i32(1)]],
                                 buf.at[pnx], gsem.at[pnx])

            pltpu.make_async_copy(h_hbm.at[src_v.at[pk, jj]],
                                  buf.at[p], gsem.at[p]).wait()
            pltpu.sync_copy(buf.at[p], acc.at[dst_v.at[pk, jj]], add=True)
            return icarry

        lax.fori_loop(i32(0), i32(CH), inner, 0)

        @pl.when(more)
        def _():
            pltpu.make_async_copy(
                src_hbm.at[pl.ds(base + (k + i32(1)) * i32(CH), CH)],
                src_v.at[pn], isem).wait()
            pltpu.make_async_copy(
                dst_hbm.at[pl.ds(base + (k + i32(1)) * i32(CH), CH)],
                dst_v.at[pn], isem).wait()
            pltpu.async_copy(h_hbm.at[src_v.at[pn, i32(0)]],
                             buf.at[i32(0)], gsem.at[i32(0)])

        return carry

    lax.fori_loop(i32(0), i32(nchunk), chunk_body, 0)


def _agg_body(h_hbm, src_hbm, dst_hbm, out0,
              src_v, dst_v, buf, acc, gsem, isem):
    c = lax.axis_index("c")
    s = lax.axis_index("s")
    i32 = jnp.int32

    @pl.when(c == 0)
    def _():
        base = s * i32(NBC0)
        pltpu.sync_copy(src_hbm.at[pl.ds(base, CH)], src_v.at[i32(0)])
        pltpu.sync_copy(dst_hbm.at[pl.ds(base, CH)], dst_v.at[i32(0)])
        rows = pl.ds(s * i32(RPT), RPT)
        pltpu.sync_copy(h_hbm.at[rows], acc.at[rows])
        pltpu.async_copy(h_hbm.at[src_v.at[i32(0), i32(0)]],
                         buf.at[i32(0)], gsem.at[i32(0)])
        plsc.subcore_barrier()
        _emit_chunks(h_hbm, src_hbm, dst_hbm, acc, src_v, dst_v, buf,
                     gsem, isem, base, NBC0 // CH)
        plsc.subcore_barrier()
        pltpu.sync_copy(acc.at[rows], out0.at[rows])


_agg_call = functools.partial(
    pl.kernel,
    out_type=jax.ShapeDtypeStruct((NP, D), jnp.float32),
    mesh=_sc_mesh(),
    scratch_types=[
        pltpu.VMEM((2, CH, BATCH), jnp.int32),
        pltpu.VMEM((2, CH, BATCH), jnp.int32),
        pltpu.VMEM((2, BATCH, D), jnp.float32),
        pltpu.VMEM_SHARED((NP, D), jnp.float32),
        pltpu.SemaphoreType.DMA((2,)),
        pltpu.SemaphoreType.DMA,
    ],
)(_agg_body)


def _tc1_body(deg0_ref, deg1_ref, x_ref, w_ref, h_ref, dinv_ref):
    d = deg0_ref[0, 0, :] + deg1_ref[0, 0, :] + 1.0
    di = lax.rsqrt(d)
    h = jnp.dot(x_ref[...], w_ref[...], preferred_element_type=jnp.float32,
                precision=lax.Precision.HIGHEST)
    h_ref[...] = di[:, None] * h
    dinv_ref[0, 0, :] = di


def _tc2_body(s0_ref, dinv_ref, b_ref, out_ref):
    di = dinv_ref[0, 0, :][:, None]
    h = jnp.maximum(di * s0_ref[...] + b_ref[...][None, :], 0.0)
    out_ref[...] = di * h


def _tc3_body(s0_ref, dinv_ref, w_ref, b_ref, out_ref):
    di = dinv_ref[0, 0, :][:, None]
    a = di * s0_ref[...]
    out_ref[...] = (
        jnp.dot(a, w_ref[...], preferred_element_type=jnp.float32,
                precision=lax.Precision.HIGHEST)
        + b_ref[...][None, :]
    )


def _row_spec(width):
    return pl.BlockSpec((BLK, width), lambda i: (i, 0))


def _vec_spec():
    return pl.BlockSpec((1, 1, BLK), lambda i: (i, 0, 0))


def _full_spec(r, c):
    return pl.BlockSpec((r, c), lambda i: (0, 0))


_tc1_call = pl.pallas_call(
    _tc1_body,
    grid=(GRID,),
    in_specs=[_vec_spec(), _vec_spec(), _row_spec(D), _full_spec(D, D)],
    out_specs=[_row_spec(D), _vec_spec()],
    out_shape=[
        jax.ShapeDtypeStruct((NP, D), jnp.float32),
        jax.ShapeDtypeStruct((GRID, 1, BLK), jnp.float32),
    ],
)

_tc2_call = pl.pallas_call(
    _tc2_body,
    grid=(GRID,),
    in_specs=[
        _row_spec(D),
        _vec_spec(),
        pl.BlockSpec((D,), lambda i: (0,)),
    ],
    out_specs=_row_spec(D),
    out_shape=jax.ShapeDtypeStruct((NP, D), jnp.float32),
)

_tc3_call = pl.pallas_call(
    _tc3_body,
    grid=(GRID,),
    in_specs=[
        _row_spec(D),
        _vec_spec(),
        _full_spec(D, D),
        pl.BlockSpec((D,), lambda i: (0,)),
    ],
    out_specs=_row_spec(D),
    out_shape=jax.ShapeDtypeStruct((NP, D), jnp.float32),
)


@jax.jit
def _run(x, src, dst, W1, b1, W_cat, b_cat):
    pad = EP - src.shape[0]
    src_p = jnp.concatenate(
        [src, jnp.zeros((pad,), jnp.int32)]).reshape(TB, BATCH)
    dst_p = jnp.concatenate(
        [dst, N + jnp.arange(pad, dtype=jnp.int32) % (NP - N)],
    ).reshape(TB, BATCH)
    xp = jnp.zeros((NP, D), jnp.float32).at[:N].set(x)

    deg0, deg1 = _deg_call(dst_p)
    hp, dinv = _tc1_call(
        deg0.reshape(GRID, 1, BLK), deg1.reshape(GRID, 1, BLK), xp, W1)
    s1 = _agg_call(hp, src_p, dst_p)
    hp2 = _tc2_call(s1, dinv, b1)
    s2 = _agg_call(hp2, src_p, dst_p)
    out = _tc3_call(s2, dinv, W_cat, b_cat)
    return out[:N, :64], out[:N, 64:]


def kernel(x, edge_index, W1, b1, W_mu, b_mu, W_var, b_var):
    with jax.enable_x64(False):
        src = edge_index[0].astype(jnp.int32)
        dst = edge_index[1].astype(jnp.int32)
        W_cat = jnp.concatenate([W_mu, W_var], axis=1)
        b_cat = jnp.concatenate([b_mu, b_var], axis=0)
        mu, lv = _run(x.astype(jnp.float32), src, dst,
                      W1.astype(jnp.float32), b1.astype(jnp.float32),
                      W_cat.astype(jnp.float32), b_cat.astype(jnp.float32))
    return mu.astype(jnp.float64), lv.astype(jnp.float64)


# speedup vs baseline: 2.7285x; 2.7264x over previous
import functools

import jax
import jax.numpy as jnp
from jax import lax
from jax.experimental import pallas as pl
from jax.experimental.pallas import tpu as pltpu
from jax.experimental.pallas import tpu_sc as plsc

N = 10000
D = 128
NC = 2
NS = 16
NW = NC * NS
NB = 80
BATCH = 128
EPW = NB * BATCH
EP = NW * EPW
NP = 10240
RPT = NP // NS
BLK = 256
GRID = NP // BLK


def _sc_mesh():
    return plsc.VectorSubcoreMesh(
        core_axis_name="c", subcore_axis_name="s",
        num_cores=NC, num_subcores=NS)


def _deg_body(dst_hbm, out0, out1, dst_v, zbuf, ones_v, acc):
    c = lax.axis_index("c")
    s = lax.axis_index("s")
    wid = c * jnp.int32(NS) + s

    def fill_z(i, carry):
        zbuf[pl.ds(i * jnp.int32(16), 16)] = jnp.zeros((16,), jnp.float32)
        return carry

    lax.fori_loop(jnp.int32(0), jnp.int32(RPT // 16), fill_z, 0)

    def fill_o(i, carry):
        ones_v[pl.ds(i * jnp.int32(16), 16)] = jnp.ones((16,), jnp.float32)
        return carry

    lax.fori_loop(jnp.int32(0), jnp.int32(BATCH // 16), fill_o, 0)

    rows = pl.ds(s * jnp.int32(RPT), RPT)
    pltpu.sync_copy(dst_hbm.at[pl.ds(wid * jnp.int32(NB), NB)], dst_v)
    pltpu.sync_copy(zbuf, acc.at[rows])
    plsc.subcore_barrier()

    def body(j, carry):
        pltpu.sync_copy(ones_v, acc.at[dst_v.at[j]], add=True)
        return carry

    lax.fori_loop(jnp.int32(0), jnp.int32(NB), body, 0)
    plsc.subcore_barrier()

    @pl.when(c == 0)
    def _():
        pltpu.sync_copy(acc.at[rows], out0.at[rows])

    @pl.when(c == 1)
    def _():
        pltpu.sync_copy(acc.at[rows], out1.at[rows])


_deg_call = functools.partial(
    pl.kernel,
    out_type=(
        jax.ShapeDtypeStruct((NP,), jnp.float32),
        jax.ShapeDtypeStruct((NP,), jnp.float32),
    ),
    mesh=_sc_mesh(),
    scratch_types=[
        pltpu.VMEM((NB, BATCH), jnp.int32),
        pltpu.VMEM((RPT,), jnp.float32),
        pltpu.VMEM((BATCH,), jnp.float32),
        pltpu.VMEM_SHARED((NP,), jnp.float32),
    ],
)(_deg_body)


CH = 16
TB = EP // BATCH
NBPT = TB // NW


def _emit_chunks(h_hbm, src_hbm, dst_hbm, acc, src_v, dst_v, buf,
                 gsem, isem, base, nchunk):
    i32 = jnp.int32

    def chunk_body(k, carry):
        pk = lax.rem(k, i32(2))
        pn = lax.rem(k + i32(1), i32(2))
        more = k + i32(1) < i32(nchunk)

        @pl.when(more)
        def _():
            pltpu.async_copy(
                src_hbm.at[pl.ds(base + (k + i32(1)) * i32(CH), CH)],
                src_v.at[pn], isem)
            pltpu.async_copy(
                dst_hbm.at[pl.ds(base + (k + i32(1)) * i32(CH), CH)],
                dst_v.at[pn], isem)

        def inner(jj, icarry):
            p = lax.rem(jj, i32(2))
            pnx = lax.rem(jj + i32(1), i32(2))

            @pl.when(jj < i32(CH - 1))
            def _():
                pltpu.async_copy(h_hbm.at[src_v.at[pk, jj + i32(1)]],
                                 buf.at[pnx], gsem.at[pnx])

            pltpu.make_async_copy(h_hbm.at[src_v.at[pk, jj]],
                                  buf.at[p], gsem.at[p]).wait()
            pltpu.sync_copy(buf.at[p], acc.at[dst_v.at[pk, jj]], add=True)
            return icarry

        lax.fori_loop(i32(0), i32(CH), inner, 0)

        @pl.when(more)
        def _():
            pltpu.make_async_copy(
                src_hbm.at[pl.ds(base + (k + i32(1)) * i32(CH), CH)],
                src_v.at[pn], isem).wait()
            pltpu.make_async_copy(
                dst_hbm.at[pl.ds(base + (k + i32(1)) * i32(CH), CH)],
                dst_v.at[pn], isem).wait()
            pltpu.async_copy(h_hbm.at[src_v.at[pn, i32(0)]],
                             buf.at[i32(0)], gsem.at[i32(0)])

        return carry

    lax.fori_loop(i32(0), i32(nchunk), chunk_body, 0)


def _agg_body(h_hbm, src_hbm, dst_hbm, out0, out1,
              src_v, dst_v, buf, acc, gsem, isem):
    c = lax.axis_index("c")
    s = lax.axis_index("s")
    i32 = jnp.int32

    base = (c * i32(NS) + s) * i32(NBPT)
    pltpu.sync_copy(src_hbm.at[pl.ds(base, CH)], src_v.at[i32(0)])
    pltpu.sync_copy(dst_hbm.at[pl.ds(base, CH)], dst_v.at[i32(0)])
    rows = pl.ds(s * i32(RPT), RPT)
    pltpu.sync_copy(h_hbm.at[rows], acc.at[rows])
    pltpu.async_copy(h_hbm.at[src_v.at[i32(0), i32(0)]],
                     buf.at[i32(0)], gsem.at[i32(0)])
    plsc.subcore_barrier()
    _emit_chunks(h_hbm, src_hbm, dst_hbm, acc, src_v, dst_v, buf,
                 gsem, isem, base, NBPT // CH)
    plsc.subcore_barrier()

    @pl.when(c == 0)
    def _():
        pltpu.sync_copy(acc.at[rows], out0.at[rows])

    @pl.when(c == 1)
    def _():
        pltpu.sync_copy(acc.at[rows], out1.at[rows])


_agg_call = functools.partial(
    pl.kernel,
    out_type=(
        jax.ShapeDtypeStruct((NP, D), jnp.float32),
        jax.ShapeDtypeStruct((NP, D), jnp.float32),
    ),
    mesh=_sc_mesh(),
    scratch_types=[
        pltpu.VMEM((2, CH, BATCH), jnp.int32),
        pltpu.VMEM((2, CH, BATCH), jnp.int32),
        pltpu.VMEM((2, BATCH, D), jnp.float32),
        pltpu.VMEM_SHARED((NP, D), jnp.float32),
        pltpu.SemaphoreType.DMA((2,)),
        pltpu.SemaphoreType.DMA,
    ],
)(_agg_body)


def _tc1_body(deg0_ref, deg1_ref, x_ref, w_ref, h_ref, dinv_ref):
    d = deg0_ref[0, 0, :] + deg1_ref[0, 0, :] + 1.0
    di = lax.rsqrt(d)
    h = jnp.dot(x_ref[...], w_ref[...], preferred_element_type=jnp.float32,
                precision=lax.Precision.HIGHEST)
    h_ref[...] = di[:, None] * h
    dinv_ref[0, 0, :] = di


def _tc2_body(s0_ref, s1_ref, hp_ref, dinv_ref, b_ref, out_ref):
    di = dinv_ref[0, 0, :][:, None]
    s = s0_ref[...] + s1_ref[...] - hp_ref[...]
    h = jnp.maximum(di * s + b_ref[...][None, :], 0.0)
    out_ref[...] = di * h


def _tc3_body(s0_ref, s1_ref, hp_ref, dinv_ref, w_ref, b_ref, out_ref):
    di = dinv_ref[0, 0, :][:, None]
    a = di * (s0_ref[...] + s1_ref[...] - hp_ref[...])
    out_ref[...] = (
        jnp.dot(a, w_ref[...], preferred_element_type=jnp.float32,
                precision=lax.Precision.HIGHEST)
        + b_ref[...][None, :]
    )


def _row_spec(width):
    return pl.BlockSpec((BLK, width), lambda i: (i, 0))


def _vec_spec():
    return pl.BlockSpec((1, 1, BLK), lambda i: (i, 0, 0))


def _full_spec(r, c):
    return pl.BlockSpec((r, c), lambda i: (0, 0))


_tc1_call = pl.pallas_call(
    _tc1_body,
    grid=(GRID,),
    in_specs=[_vec_spec(), _vec_spec(), _row_spec(D), _full_spec(D, D)],
    out_specs=[_row_spec(D), _vec_spec()],
    out_shape=[
        jax.ShapeDtypeStruct((NP, D), jnp.float32),
        jax.ShapeDtypeStruct((GRID, 1, BLK), jnp.float32),
    ],
)

_tc2_call = pl.pallas_call(
    _tc2_body,
    grid=(GRID,),
    in_specs=[
        _row_spec(D),
        _row_spec(D),
        _row_spec(D),
        _vec_spec(),
        pl.BlockSpec((D,), lambda i: (0,)),
    ],
    out_specs=_row_spec(D),
    out_shape=jax.ShapeDtypeStruct((NP, D), jnp.float32),
)

_tc3_call = pl.pallas_call(
    _tc3_body,
    grid=(GRID,),
    in_specs=[
        _row_spec(D),
        _row_spec(D),
        _row_spec(D),
        _vec_spec(),
        _full_spec(D, D),
        pl.BlockSpec((D,), lambda i: (0,)),
    ],
    out_specs=_row_spec(D),
    out_shape=jax.ShapeDtypeStruct((NP, D), jnp.float32),
)


@jax.jit
def _run(x, src, dst, W1, b1, W_cat, b_cat):
    pad = EP - src.shape[0]
    src_p = jnp.concatenate(
        [src, jnp.arange(pad, dtype=jnp.int32) % N]).reshape(TB, BATCH)
    dst_p = jnp.concatenate(
        [dst, N + jnp.arange(pad, dtype=jnp.int32) % (NP - N)],
    ).reshape(TB, BATCH)
    xp = jnp.zeros((NP, D), jnp.float32).at[:N].set(x)

    deg0, deg1 = _deg_call(dst_p)
    hp, dinv = _tc1_call(
        deg0.reshape(GRID, 1, BLK), deg1.reshape(GRID, 1, BLK), xp, W1)
    s1a, s1b = _agg_call(hp, src_p, dst_p)
    hp2 = _tc2_call(s1a, s1b, hp, dinv, b1)
    s2a, s2b = _agg_call(hp2, src_p, dst_p)
    out = _tc3_call(s2a, s2b, hp2, dinv, W_cat, b_cat)
    return out[:N, :64], out[:N, 64:]


def kernel(x, edge_index, W1, b1, W_mu, b_mu, W_var, b_var):
    with jax.enable_x64(False):
        src = edge_index[0].astype(jnp.int32)
        dst = edge_index[1].astype(jnp.int32)
        W_cat = jnp.concatenate([W_mu, W_var], axis=1)
        b_cat = jnp.concatenate([b_mu, b_var], axis=0)
        mu, lv = _run(x.astype(jnp.float32), src, dst,
                      W1.astype(jnp.float32), b1.astype(jnp.float32),
                      W_cat.astype(jnp.float32), b_cat.astype(jnp.float32))
    return mu.astype(jnp.float64), lv.astype(jnp.float64)


# speedup vs baseline: 2.9253x; 1.0721x over previous
import functools

import jax
import jax.numpy as jnp
from jax import lax
from jax.experimental import pallas as pl
from jax.experimental.pallas import tpu as pltpu
from jax.experimental.pallas import tpu_sc as plsc

N = 10000
D = 128
NC = 2
NS = 16
NW = NC * NS
NB = 80
BATCH = 128
EPW = NB * BATCH
EP = NW * EPW
NP = 10240
RPT = NP // NS
BLK = 512
GRID = NP // BLK


def _sc_mesh():
    return plsc.VectorSubcoreMesh(
        core_axis_name="c", subcore_axis_name="s",
        num_cores=NC, num_subcores=NS)


def _deg_body(dst_hbm, out0, out1, dst_v, zbuf, ones_v, acc):
    c = lax.axis_index("c")
    s = lax.axis_index("s")
    wid = c * jnp.int32(NS) + s

    def fill_z(i, carry):
        zbuf[pl.ds(i * jnp.int32(16), 16)] = jnp.zeros((16,), jnp.float32)
        return carry

    lax.fori_loop(jnp.int32(0), jnp.int32(RPT // 16), fill_z, 0)

    def fill_o(i, carry):
        ones_v[pl.ds(i * jnp.int32(16), 16)] = jnp.ones((16,), jnp.float32)
        return carry

    lax.fori_loop(jnp.int32(0), jnp.int32(BATCH // 16), fill_o, 0)

    rows = pl.ds(s * jnp.int32(RPT), RPT)
    pltpu.sync_copy(dst_hbm.at[pl.ds(wid * jnp.int32(NB), NB)], dst_v)
    pltpu.sync_copy(zbuf, acc.at[rows])
    plsc.subcore_barrier()

    def body(j, carry):
        pltpu.sync_copy(ones_v, acc.at[dst_v.at[j]], add=True)
        return carry

    lax.fori_loop(jnp.int32(0), jnp.int32(NB), body, 0)
    plsc.subcore_barrier()

    @pl.when(c == 0)
    def _():
        pltpu.sync_copy(acc.at[rows], out0.at[rows])

    @pl.when(c == 1)
    def _():
        pltpu.sync_copy(acc.at[rows], out1.at[rows])


_deg_call = functools.partial(
    pl.kernel,
    out_type=(
        jax.ShapeDtypeStruct((NP,), jnp.float32),
        jax.ShapeDtypeStruct((NP,), jnp.float32),
    ),
    mesh=_sc_mesh(),
    scratch_types=[
        pltpu.VMEM((NB, BATCH), jnp.int32),
        pltpu.VMEM((RPT,), jnp.float32),
        pltpu.VMEM((BATCH,), jnp.float32),
        pltpu.VMEM_SHARED((NP,), jnp.float32),
    ],
)(_deg_body)


CH = 16
TB = EP // BATCH
NBPT = TB // NW


def _emit_chunks(h_hbm, src_hbm, dst_hbm, acc, src_v, dst_v, buf,
                 gsem, isem, base, nchunk):
    i32 = jnp.int32

    def chunk_body(k, carry):
        pk = lax.rem(k, i32(2))
        pn = lax.rem(k + i32(1), i32(2))
        more = k + i32(1) < i32(nchunk)

        @pl.when(more)
        def _():
            pltpu.async_copy(
                src_hbm.at[pl.ds(base + (k + i32(1)) * i32(CH), CH)],
                src_v.at[pn], isem)
            pltpu.async_copy(
                dst_hbm.at[pl.ds(base + (k + i32(1)) * i32(CH), CH)],
                dst_v.at[pn], isem)

        def inner(jj, icarry):
            p = lax.rem(jj, i32(2))
            pnx = lax.rem(jj + i32(1), i32(2))

            @pl.when(jj < i32(CH - 1))
            def _():
                pltpu.async_copy(h_hbm.at[src_v.at[pk, jj + i32(1)]],
                                 buf.at[pnx], gsem.at[pnx])

            pltpu.make_async_copy(h_hbm.at[src_v.at[pk, jj]],
                                  buf.at[p], gsem.at[p]).wait()
            pltpu.sync_copy(buf.at[p], acc.at[dst_v.at[pk, jj]], add=True)
            return icarry

        lax.fori_loop(i32(0), i32(CH), inner, 0)

        @pl.when(more)
        def _():
            pltpu.make_async_copy(
                src_hbm.at[pl.ds(base + (k + i32(1)) * i32(CH), CH)],
                src_v.at[pn], isem).wait()
            pltpu.make_async_copy(
                dst_hbm.at[pl.ds(base + (k + i32(1)) * i32(CH), CH)],
                dst_v.at[pn], isem).wait()
            pltpu.async_copy(h_hbm.at[src_v.at[pn, i32(0)]],
                             buf.at[i32(0)], gsem.at[i32(0)])

        return carry

    lax.fori_loop(i32(0), i32(nchunk), chunk_body, 0)


def _agg_body(h_hbm, src_hbm, dst_hbm, out0, out1,
              src_v, dst_v, buf, acc, gsem, isem):
    c = lax.axis_index("c")
    s = lax.axis_index("s")
    i32 = jnp.int32

    base = (c * i32(NS) + s) * i32(NBPT)
    rows = pl.ds(s * i32(RPT), RPT)

    @pl.when(c == 0)
    def _():
        pltpu.sync_copy(h_hbm.at[rows], acc.at[rows])

    @pl.when(c == 1)
    def _():
        def fill_z(i, carry):
            for k in range(D // 16):
                buf[i32(0), i, pl.ds(i32(16 * k), 16)] = jnp.zeros(
                    (16,), jnp.float32)
            return carry

        lax.fori_loop(i32(0), i32(BATCH), fill_z, 0)
        for r in range(RPT // BATCH):
            pltpu.sync_copy(
                buf.at[i32(0)],
                acc.at[pl.ds(s * i32(RPT) + i32(BATCH * r), BATCH)])

    pltpu.sync_copy(src_hbm.at[pl.ds(base, CH)], src_v.at[i32(0)])
    pltpu.sync_copy(dst_hbm.at[pl.ds(base, CH)], dst_v.at[i32(0)])
    pltpu.async_copy(h_hbm.at[src_v.at[i32(0), i32(0)]],
                     buf.at[i32(0)], gsem.at[i32(0)])
    plsc.subcore_barrier()
    _emit_chunks(h_hbm, src_hbm, dst_hbm, acc, src_v, dst_v, buf,
                 gsem, isem, base, NBPT // CH)
    plsc.subcore_barrier()

    @pl.when(c == 0)
    def _():
        pltpu.sync_copy(acc.at[rows], out0.at[rows])

    @pl.when(c == 1)
    def _():
        pltpu.sync_copy(acc.at[rows], out1.at[rows])


_agg_call = functools.partial(
    pl.kernel,
    out_type=(
        jax.ShapeDtypeStruct((NP, D), jnp.float32),
        jax.ShapeDtypeStruct((NP, D), jnp.float32),
    ),
    mesh=_sc_mesh(),
    scratch_types=[
        pltpu.VMEM((2, CH, BATCH), jnp.int32),
        pltpu.VMEM((2, CH, BATCH), jnp.int32),
        pltpu.VMEM((2, BATCH, D), jnp.float32),
        pltpu.VMEM_SHARED((NP, D), jnp.float32),
        pltpu.SemaphoreType.DMA((2,)),
        pltpu.SemaphoreType.DMA,
    ],
)(_agg_body)


def _tc1_body(deg0_ref, deg1_ref, x_ref, w_ref, h_ref, dinv_ref):
    d = deg0_ref[0, 0, :] + deg1_ref[0, 0, :] + 1.0
    di = lax.rsqrt(d)
    h = jnp.dot(x_ref[...], w_ref[...], preferred_element_type=jnp.float32,
                precision=lax.Precision.HIGHEST)
    h_ref[...] = di[:, None] * h
    dinv_ref[0, 0, :] = di


def _tc2_body(s0_ref, s1_ref, dinv_ref, b_ref, out_ref):
    di = dinv_ref[0, 0, :][:, None]
    s = s0_ref[...] + s1_ref[...]
    h = jnp.maximum(di * s + b_ref[...][None, :], 0.0)
    out_ref[...] = di * h


def _tc3_body(s0_ref, s1_ref, dinv_ref, w_ref, b_ref, out_ref):
    di = dinv_ref[0, 0, :][:, None]
    a = di * (s0_ref[...] + s1_ref[...])
    out_ref[...] = (
        jnp.dot(a, w_ref[...], preferred_element_type=jnp.float32,
                precision=lax.Precision.HIGHEST)
        + b_ref[...][None, :]
    )


def _row_spec(width):
    return pl.BlockSpec((BLK, width), lambda i: (i, 0))


def _vec_spec():
    return pl.BlockSpec((1, 1, BLK), lambda i: (i, 0, 0))


def _full_spec(r, c):
    return pl.BlockSpec((r, c), lambda i: (0, 0))


_tc1_call = pl.pallas_call(
    _tc1_body,
    grid=(GRID,),
    in_specs=[_vec_spec(), _vec_spec(), _row_spec(D), _full_spec(D, D)],
    out_specs=[_row_spec(D), _vec_spec()],
    out_shape=[
        jax.ShapeDtypeStruct((NP, D), jnp.float32),
        jax.ShapeDtypeStruct((GRID, 1, BLK), jnp.float32),
    ],
)

_tc2_call = pl.pallas_call(
    _tc2_body,
    grid=(GRID,),
    in_specs=[
        _row_spec(D),
        _row_spec(D),
        _vec_spec(),
        pl.BlockSpec((D,), lambda i: (0,)),
    ],
    out_specs=_row_spec(D),
    out_shape=jax.ShapeDtypeStruct((NP, D), jnp.float32),
)

_tc3_call = pl.pallas_call(
    _tc3_body,
    grid=(GRID,),
    in_specs=[
        _row_spec(D),
        _row_spec(D),
        _vec_spec(),
        _full_spec(D, D),
        pl.BlockSpec((D,), lambda i: (0,)),
    ],
    out_specs=_row_spec(D),
    out_shape=jax.ShapeDtypeStruct((NP, D), jnp.float32),
)


@jax.jit
def _run(x, src, dst, W1, b1, W_cat, b_cat):
    pad = EP - src.shape[0]
    src_p = jnp.concatenate(
        [src, jnp.arange(pad, dtype=jnp.int32) % N]).reshape(TB, BATCH)
    dst_p = jnp.concatenate(
        [dst, N + jnp.arange(pad, dtype=jnp.int32) % (NP - N)],
    ).reshape(TB, BATCH)
    xp = jnp.zeros((NP, D), jnp.float32).at[:N].set(x)

    deg0, deg1 = _deg_call(dst_p)
    hp, dinv = _tc1_call(
        deg0.reshape(GRID, 1, BLK), deg1.reshape(GRID, 1, BLK), xp, W1)
    s1a, s1b = _agg_call(hp, src_p, dst_p)
    hp2 = _tc2_call(s1a, s1b, dinv, b1)
    s2a, s2b = _agg_call(hp2, src_p, dst_p)
    out = _tc3_call(s2a, s2b, dinv, W_cat, b_cat)
    return out[:N, :64], out[:N, 64:]


def kernel(x, edge_index, W1, b1, W_mu, b_mu, W_var, b_var):
    with jax.enable_x64(False):
        src = edge_index[0].astype(jnp.int32)
        dst = edge_index[1].astype(jnp.int32)
        W_cat = jnp.concatenate([W_mu, W_var], axis=1)
        b_cat = jnp.concatenate([b_mu, b_var], axis=0)
        mu, lv = _run(x.astype(jnp.float32), src, dst,
                      W1.astype(jnp.float32), b1.astype(jnp.float32),
                      W_cat.astype(jnp.float32), b_cat.astype(jnp.float32))
    return mu.astype(jnp.float64), lv.astype(jnp.float64)


# speedup vs baseline: 3.0246x; 1.0339x over previous
import functools

import jax
import jax.numpy as jnp
from jax import lax
from jax.experimental import pallas as pl
from jax.experimental.pallas import tpu as pltpu
from jax.experimental.pallas import tpu_sc as plsc

N = 10000
D = 128
NC = 2
NS = 16
NW = NC * NS
NB = 80
BATCH = 128
EPW = NB * BATCH
EP = NW * EPW
NP = 10240
RPT = NP // NS
BLK = 1024
GRID = NP // BLK


def _sc_mesh():
    return plsc.VectorSubcoreMesh(
        core_axis_name="c", subcore_axis_name="s",
        num_cores=NC, num_subcores=NS)


def _deg_body(dst_hbm, out0, out1, dst_v, zbuf, ones_v, acc):
    c = lax.axis_index("c")
    s = lax.axis_index("s")
    wid = c * jnp.int32(NS) + s

    def fill_z(i, carry):
        zbuf[pl.ds(i * jnp.int32(16), 16)] = jnp.zeros((16,), jnp.float32)
        return carry

    lax.fori_loop(jnp.int32(0), jnp.int32(RPT // 16), fill_z, 0)

    def fill_o(i, carry):
        ones_v[pl.ds(i * jnp.int32(16), 16)] = jnp.ones((16,), jnp.float32)
        return carry

    lax.fori_loop(jnp.int32(0), jnp.int32(BATCH // 16), fill_o, 0)

    rows = pl.ds(s * jnp.int32(RPT), RPT)
    pltpu.sync_copy(dst_hbm.at[pl.ds(wid * jnp.int32(NB), NB)], dst_v)
    pltpu.sync_copy(zbuf, acc.at[rows])
    plsc.subcore_barrier()

    def body(j, carry):
        pltpu.sync_copy(ones_v, acc.at[dst_v.at[j]], add=True)
        return carry

    lax.fori_loop(jnp.int32(0), jnp.int32(NB), body, 0)
    plsc.subcore_barrier()

    @pl.when(c == 0)
    def _():
        pltpu.sync_copy(acc.at[rows], out0.at[rows])

    @pl.when(c == 1)
    def _():
        pltpu.sync_copy(acc.at[rows], out1.at[rows])


_deg_call = functools.partial(
    pl.kernel,
    out_type=(
        jax.ShapeDtypeStruct((NP,), jnp.float32),
        jax.ShapeDtypeStruct((NP,), jnp.float32),
    ),
    mesh=_sc_mesh(),
    scratch_types=[
        pltpu.VMEM((NB, BATCH), jnp.int32),
        pltpu.VMEM((RPT,), jnp.float32),
        pltpu.VMEM((BATCH,), jnp.float32),
        pltpu.VMEM_SHARED((NP,), jnp.float32),
    ],
)(_deg_body)


CH = 16
TB = EP // BATCH
NBPT = TB // NW


def _emit_chunks(h_hbm, src_hbm, dst_hbm, acc, src_v, dst_v, buf,
                 gsem, isem, base, nchunk):
    i32 = jnp.int32

    def chunk_body(k, carry):
        pk = lax.rem(k, i32(2))
        pn = lax.rem(k + i32(1), i32(2))
        more = k + i32(1) < i32(nchunk)

        @pl.when(more)
        def _():
            pltpu.async_copy(
                src_hbm.at[pl.ds(base + (k + i32(1)) * i32(CH), CH)],
                src_v.at[pn], isem)
            pltpu.async_copy(
                dst_hbm.at[pl.ds(base + (k + i32(1)) * i32(CH), CH)],
                dst_v.at[pn], isem)

        def inner(jj, icarry):
            p = lax.rem(jj, i32(2))
            pnx = lax.rem(jj + i32(1), i32(2))

            @pl.when(jj < i32(CH - 1))
            def _():
                pltpu.async_copy(h_hbm.at[src_v.at[pk, jj + i32(1)]],
                                 buf.at[pnx], gsem.at[pnx])

            pltpu.make_async_copy(h_hbm.at[src_v.at[pk, jj]],
                                  buf.at[p], gsem.at[p]).wait()
            pltpu.sync_copy(buf.at[p], acc.at[dst_v.at[pk, jj]], add=True)
            return icarry

        lax.fori_loop(i32(0), i32(CH), inner, 0)

        @pl.when(more)
        def _():
            pltpu.make_async_copy(
                src_hbm.at[pl.ds(base + (k + i32(1)) * i32(CH), CH)],
                src_v.at[pn], isem).wait()
            pltpu.make_async_copy(
                dst_hbm.at[pl.ds(base + (k + i32(1)) * i32(CH), CH)],
                dst_v.at[pn], isem).wait()
            pltpu.async_copy(h_hbm.at[src_v.at[pn, i32(0)]],
                             buf.at[i32(0)], gsem.at[i32(0)])

        return carry

    lax.fori_loop(i32(0), i32(nchunk), chunk_body, 0)


def _agg_body(h_hbm, src_hbm, dst_hbm, out0, out1,
              src_v, dst_v, buf, acc, gsem, isem):
    c = lax.axis_index("c")
    s = lax.axis_index("s")
    i32 = jnp.int32

    base = (c * i32(NS) + s) * i32(NBPT)
    rows = pl.ds(s * i32(RPT), RPT)

    @pl.when(c == 0)
    def _():
        pltpu.sync_copy(h_hbm.at[rows], acc.at[rows])

    @pl.when(c == 1)
    def _():
        def fill_z(i, carry):
            for k in range(D // 16):
                buf[i32(0), i, pl.ds(i32(16 * k), 16)] = jnp.zeros(
                    (16,), jnp.float32)
            return carry

        lax.fori_loop(i32(0), i32(BATCH), fill_z, 0)
        for r in range(RPT // BATCH):
            pltpu.sync_copy(
                buf.at[i32(0)],
                acc.at[pl.ds(s * i32(RPT) + i32(BATCH * r), BATCH)])

    pltpu.sync_copy(src_hbm.at[pl.ds(base, CH)], src_v.at[i32(0)])
    pltpu.sync_copy(dst_hbm.at[pl.ds(base, CH)], dst_v.at[i32(0)])
    pltpu.async_copy(h_hbm.at[src_v.at[i32(0), i32(0)]],
                     buf.at[i32(0)], gsem.at[i32(0)])
    plsc.subcore_barrier()
    _emit_chunks(h_hbm, src_hbm, dst_hbm, acc, src_v, dst_v, buf,
                 gsem, isem, base, NBPT // CH)
    plsc.subcore_barrier()

    @pl.when(c == 0)
    def _():
        pltpu.sync_copy(acc.at[rows], out0.at[rows])

    @pl.when(c == 1)
    def _():
        pltpu.sync_copy(acc.at[rows], out1.at[rows])


_agg_call = functools.partial(
    pl.kernel,
    out_type=(
        jax.ShapeDtypeStruct((NP, D), jnp.float32),
        jax.ShapeDtypeStruct((NP, D), jnp.float32),
    ),
    mesh=_sc_mesh(),
    scratch_types=[
        pltpu.VMEM((2, CH, BATCH), jnp.int32),
        pltpu.VMEM((2, CH, BATCH), jnp.int32),
        pltpu.VMEM((2, BATCH, D), jnp.float32),
        pltpu.VMEM_SHARED((NP, D), jnp.float32),
        pltpu.SemaphoreType.DMA((2,)),
        pltpu.SemaphoreType.DMA,
    ],
)(_agg_body)


def _tc1_body(deg0_ref, deg1_ref, x_ref, w_ref, h_ref, dinv_ref):
    d = deg0_ref[0, 0, :] + deg1_ref[0, 0, :] + 1.0
    di = lax.rsqrt(d)
    h = jnp.dot(x_ref[...], w_ref[...], preferred_element_type=jnp.float32,
                precision=lax.Precision.HIGHEST)
    h_ref[...] = di[:, None] * h
    dinv_ref[0, 0, :] = di


def _tc2_body(s0_ref, s1_ref, dinv_ref, b_ref, out_ref):
    di = dinv_ref[0, 0, :][:, None]
    s = s0_ref[...] + s1_ref[...]
    h = jnp.maximum(di * s + b_ref[...][None, :], 0.0)
    out_ref[...] = di * h


def _tc3_body(s0_ref, s1_ref, dinv_ref, w_ref, b_ref, out_ref):
    di = dinv_ref[0, 0, :][:, None]
    a = di * (s0_ref[...] + s1_ref[...])
    out_ref[...] = (
        jnp.dot(a, w_ref[...], preferred_element_type=jnp.float32,
                precision=lax.Precision.HIGHEST)
        + b_ref[...][None, :]
    )


def _row_spec(width):
    return pl.BlockSpec((BLK, width), lambda i: (i, 0))


def _vec_spec():
    return pl.BlockSpec((1, 1, BLK), lambda i: (i, 0, 0))


def _full_spec(r, c):
    return pl.BlockSpec((r, c), lambda i: (0, 0))


_tc1_call = pl.pallas_call(
    _tc1_body,
    grid=(GRID,),
    in_specs=[_vec_spec(), _vec_spec(), _row_spec(D), _full_spec(D, D)],
    out_specs=[_row_spec(D), _vec_spec()],
    out_shape=[
        jax.ShapeDtypeStruct((NP, D), jnp.float32),
        jax.ShapeDtypeStruct((GRID, 1, BLK), jnp.float32),
    ],
)

_tc2_call = pl.pallas_call(
    _tc2_body,
    grid=(GRID,),
    in_specs=[
        _row_spec(D),
        _row_spec(D),
        _vec_spec(),
        pl.BlockSpec((D,), lambda i: (0,)),
    ],
    out_specs=_row_spec(D),
    out_shape=jax.ShapeDtypeStruct((NP, D), jnp.float32),
)

_tc3_call = pl.pallas_call(
    _tc3_body,
    grid=(GRID,),
    in_specs=[
        _row_spec(D),
        _row_spec(D),
        _vec_spec(),
        _full_spec(D, D),
        pl.BlockSpec((D,), lambda i: (0,)),
    ],
    out_specs=_row_spec(D),
    out_shape=jax.ShapeDtypeStruct((NP, D), jnp.float32),
)


@jax.jit
def _run(x, src, dst, W1, b1, W_cat, b_cat):
    pad = EP - src.shape[0]
    src_p = jnp.concatenate(
        [src, jnp.arange(pad, dtype=jnp.int32) % N]).reshape(TB, BATCH)
    dst_p = jnp.concatenate(
        [dst, N + jnp.arange(pad, dtype=jnp.int32) % (NP - N)],
    ).reshape(TB, BATCH)
    xp = jnp.zeros((NP, D), jnp.float32).at[:N].set(x)

    deg0, deg1 = _deg_call(dst_p)
    hp, dinv = _tc1_call(
        deg0.reshape(GRID, 1, BLK), deg1.reshape(GRID, 1, BLK), xp, W1)
    s1a, s1b = _agg_call(hp, src_p, dst_p)
    hp2 = _tc2_call(s1a, s1b, dinv, b1)
    s2a, s2b = _agg_call(hp2, src_p, dst_p)
    out = _tc3_call(s2a, s2b, dinv, W_cat, b_cat)
    return out[:N, :64], out[:N, 64:]


@jax.jit
def _post(mu, lv):
    return mu.astype(jnp.float64), lv.astype(jnp.float64)


def kernel(x, edge_index, W1, b1, W_mu, b_mu, W_var, b_var):
    with jax.enable_x64(False):
        src = edge_index[0].astype(jnp.int32)
        dst = edge_index[1].astype(jnp.int32)
        W_cat = jnp.concatenate([W_mu, W_var], axis=1)
        b_cat = jnp.concatenate([b_mu, b_var], axis=0)
        mu, lv = _run(x.astype(jnp.float32), src, dst,
                      W1.astype(jnp.float32), b1.astype(jnp.float32),
                      W_cat.astype(jnp.float32), b_cat.astype(jnp.float32))
    return _post(mu, lv)


# speedup vs baseline: 3.0842x; 1.0197x over previous
import functools

import jax
import jax.numpy as jnp
from jax import lax
from jax.experimental import pallas as pl
from jax.experimental.pallas import tpu as pltpu
from jax.experimental.pallas import tpu_sc as plsc

N = 10000
D = 128
NC = 2
NS = 16
NW = NC * NS
NB = 80
BATCH = 128
EPW = NB * BATCH
EP = NW * EPW
NP = 10240
RPT = NP // NS
BLK = 1024
GRID = NP // BLK


def _sc_mesh():
    return plsc.VectorSubcoreMesh(
        core_axis_name="c", subcore_axis_name="s",
        num_cores=NC, num_subcores=NS)


def _deg_body(dst_hbm, out0, out1, dst_v, zbuf, ones_v, acc):
    c = lax.axis_index("c")
    s = lax.axis_index("s")
    wid = c * jnp.int32(NS) + s

    def fill_z(i, carry):
        zbuf[pl.ds(i * jnp.int32(16), 16)] = jnp.zeros((16,), jnp.float32)
        return carry

    lax.fori_loop(jnp.int32(0), jnp.int32(RPT // 16), fill_z, 0)

    def fill_o(i, carry):
        ones_v[pl.ds(i * jnp.int32(16), 16)] = jnp.ones((16,), jnp.float32)
        return carry

    lax.fori_loop(jnp.int32(0), jnp.int32(BATCH // 16), fill_o, 0)

    rows = pl.ds(s * jnp.int32(RPT), RPT)
    pltpu.sync_copy(dst_hbm.at[pl.ds(wid * jnp.int32(NB), NB)], dst_v)
    pltpu.sync_copy(zbuf, acc.at[rows])
    plsc.subcore_barrier()

    def body(j, carry):
        pltpu.sync_copy(ones_v, acc.at[dst_v.at[j]], add=True)
        return carry

    lax.fori_loop(jnp.int32(0), jnp.int32(NB), body, 0)
    plsc.subcore_barrier()

    @pl.when(c == 0)
    def _():
        pltpu.sync_copy(acc.at[rows], out0.at[rows])

    @pl.when(c == 1)
    def _():
        pltpu.sync_copy(acc.at[rows], out1.at[rows])


_deg_call = functools.partial(
    pl.kernel,
    out_type=(
        jax.ShapeDtypeStruct((NP,), jnp.float32),
        jax.ShapeDtypeStruct((NP,), jnp.float32),
    ),
    mesh=_sc_mesh(),
    scratch_types=[
        pltpu.VMEM((NB, BATCH), jnp.int32),
        pltpu.VMEM((RPT,), jnp.float32),
        pltpu.VMEM((BATCH,), jnp.float32),
        pltpu.VMEM_SHARED((NP,), jnp.float32),
    ],
)(_deg_body)


CH = 16
TB = EP // BATCH
NBPT = TB // NW


def _emit_chunks(h_hbm, src_hbm, dst_hbm, acc, src_v, dst_v, buf,
                 gsem, isem, base, nchunk):
    i32 = jnp.int32

    def chunk_body(k, carry):
        pk = lax.rem(k, i32(2))
        pn = lax.rem(k + i32(1), i32(2))
        more = k + i32(1) < i32(nchunk)

        @pl.when(more)
        def _():
            pltpu.async_copy(
                src_hbm.at[pl.ds(base + (k + i32(1)) * i32(CH), CH)],
                src_v.at[pn], isem)
            pltpu.async_copy(
                dst_hbm.at[pl.ds(base + (k + i32(1)) * i32(CH), CH)],
                dst_v.at[pn], isem)

        def inner(jj, icarry):
            p = lax.rem(jj, i32(2))
            pnx = lax.rem(jj + i32(1), i32(2))

            @pl.when(jj < i32(CH - 1))
            def _():
                pltpu.async_copy(h_hbm.at[src_v.at[pk, jj + i32(1)]],
                                 buf.at[pnx], gsem.at[pnx])

            pltpu.make_async_copy(h_hbm.at[src_v.at[pk, jj]],
                                  buf.at[p], gsem.at[p]).wait()
            pltpu.sync_copy(buf.at[p], acc.at[dst_v.at[pk, jj]], add=True)
            return icarry

        lax.fori_loop(i32(0), i32(CH), inner, 0)

        @pl.when(more)
        def _():
            pltpu.make_async_copy(
                src_hbm.at[pl.ds(base + (k + i32(1)) * i32(CH), CH)],
                src_v.at[pn], isem).wait()
            pltpu.make_async_copy(
                dst_hbm.at[pl.ds(base + (k + i32(1)) * i32(CH), CH)],
                dst_v.at[pn], isem).wait()
            pltpu.async_copy(h_hbm.at[src_v.at[pn, i32(0)]],
                             buf.at[i32(0)], gsem.at[i32(0)])

        return carry

    lax.fori_loop(i32(0), i32(nchunk), chunk_body, 0)


def _agg_body(h_hbm, src_hbm, dst_hbm, out0, out1,
              src_v, dst_v, buf, acc, gsem, isem):
    c = lax.axis_index("c")
    s = lax.axis_index("s")
    i32 = jnp.int32

    base = (c * i32(NS) + s) * i32(NBPT)
    rows = pl.ds(s * i32(RPT), RPT)

    @pl.when(c == 0)
    def _():
        pltpu.sync_copy(h_hbm.at[rows], acc.at[rows])

    @pl.when(c == 1)
    def _():
        def fill_z(i, carry):
            for k in range(D // 16):
                buf[i32(0), i, pl.ds(i32(16 * k), 16)] = jnp.zeros(
                    (16,), jnp.float32)
            return carry

        lax.fori_loop(i32(0), i32(BATCH), fill_z, 0)
        for r in range(RPT // BATCH):
            pltpu.sync_copy(
                buf.at[i32(0)],
                acc.at[pl.ds(s * i32(RPT) + i32(BATCH * r), BATCH)])

    pltpu.sync_copy(src_hbm.at[pl.ds(base, CH)], src_v.at[i32(0)])
    pltpu.sync_copy(dst_hbm.at[pl.ds(base, CH)], dst_v.at[i32(0)])
    pltpu.async_copy(h_hbm.at[src_v.at[i32(0), i32(0)]],
                     buf.at[i32(0)], gsem.at[i32(0)])
    plsc.subcore_barrier()
    _emit_chunks(h_hbm, src_hbm, dst_hbm, acc, src_v, dst_v, buf,
                 gsem, isem, base, NBPT // CH)
    plsc.subcore_barrier()

    @pl.when(c == 0)
    def _():
        pltpu.sync_copy(acc.at[rows], out0.at[rows])

    @pl.when(c == 1)
    def _():
        pltpu.sync_copy(acc.at[rows], out1.at[rows])


_agg_call = functools.partial(
    pl.kernel,
    out_type=(
        jax.ShapeDtypeStruct((NP, D), jnp.float32),
        jax.ShapeDtypeStruct((NP, D), jnp.float32),
    ),
    mesh=_sc_mesh(),
    scratch_types=[
        pltpu.VMEM((2, CH, BATCH), jnp.int32),
        pltpu.VMEM((2, CH, BATCH), jnp.int32),
        pltpu.VMEM((2, BATCH, D), jnp.float32),
        pltpu.VMEM_SHARED((NP, D), jnp.float32),
        pltpu.SemaphoreType.DMA((2,)),
        pltpu.SemaphoreType.DMA,
    ],
)(_agg_body)


def _tc1_body(deg0_ref, deg1_ref, x_ref, w_ref, h_ref, dinv_ref):
    d = deg0_ref[0, 0, :] + deg1_ref[0, 0, :] + 1.0
    di = lax.rsqrt(d)
    h = jnp.dot(x_ref[...], w_ref[...], preferred_element_type=jnp.float32,
                precision=lax.Precision.HIGHEST)
    h_ref[...] = di[:, None] * h
    dinv_ref[0, 0, :] = di


def _tc2_body(s0_ref, s1_ref, dinv_ref, b_ref, out_ref):
    di = dinv_ref[0, 0, :][:, None]
    s = s0_ref[...] + s1_ref[...]
    h = jnp.maximum(di * s + b_ref[...][None, :], 0.0)
    out_ref[...] = di * h


def _tc3_body(s0_ref, s1_ref, dinv_ref, w_ref, b_ref, lo_ref, hi_ref):
    di = dinv_ref[0, 0, :][:, None]
    a = di * (s0_ref[...] + s1_ref[...])
    y = (
        jnp.dot(a, w_ref[...], preferred_element_type=jnp.float32,
                precision=lax.Precision.HIGHEST)
        + b_ref[...][None, :]
    )
    bits = lax.bitcast_convert_type(y, jnp.uint32)
    sign = bits & jnp.uint32(0x80000000)
    exp8 = (bits >> jnp.uint32(23)) & jnp.uint32(0xFF)
    mant = bits & jnp.uint32(0x7FFFFF)
    exp11 = jnp.where(exp8 == jnp.uint32(255), jnp.uint32(2047),
                      exp8 + jnp.uint32(896))
    hi = sign | (exp11 << jnp.uint32(20)) | (mant >> jnp.uint32(3))
    zero = exp8 == jnp.uint32(0)
    hi_ref[...] = jnp.where(zero, sign, hi)
    lo_ref[...] = jnp.where(zero, jnp.uint32(0), mant << jnp.uint32(29))


def _row_spec(width):
    return pl.BlockSpec((BLK, width), lambda i: (i, 0))


def _vec_spec():
    return pl.BlockSpec((1, 1, BLK), lambda i: (i, 0, 0))


def _full_spec(r, c):
    return pl.BlockSpec((r, c), lambda i: (0, 0))


_tc1_call = pl.pallas_call(
    _tc1_body,
    grid=(GRID,),
    in_specs=[_vec_spec(), _vec_spec(), _row_spec(D), _full_spec(D, D)],
    out_specs=[_row_spec(D), _vec_spec()],
    out_shape=[
        jax.ShapeDtypeStruct((NP, D), jnp.float32),
        jax.ShapeDtypeStruct((GRID, 1, BLK), jnp.float32),
    ],
)

_tc2_call = pl.pallas_call(
    _tc2_body,
    grid=(GRID,),
    in_specs=[
        _row_spec(D),
        _row_spec(D),
        _vec_spec(),
        pl.BlockSpec((D,), lambda i: (0,)),
    ],
    out_specs=_row_spec(D),
    out_shape=jax.ShapeDtypeStruct((NP, D), jnp.float32),
)

_tc3_call = pl.pallas_call(
    _tc3_body,
    grid=(GRID,),
    in_specs=[
        _row_spec(D),
        _row_spec(D),
        _vec_spec(),
        _full_spec(D, D),
        pl.BlockSpec((D,), lambda i: (0,)),
    ],
    out_specs=[_row_spec(D), _row_spec(D)],
    out_shape=[
        jax.ShapeDtypeStruct((NP, D), jnp.uint32),
        jax.ShapeDtypeStruct((NP, D), jnp.uint32),
    ],
)


@jax.jit
def _run(x, src, dst, W1, b1, W_cat, b_cat):
    pad = EP - src.shape[0]
    src_p = jnp.concatenate(
        [src, jnp.arange(pad, dtype=jnp.int32) % N]).reshape(TB, BATCH)
    dst_p = jnp.concatenate(
        [dst, N + jnp.arange(pad, dtype=jnp.int32) % (NP - N)],
    ).reshape(TB, BATCH)
    xp = jnp.zeros((NP, D), jnp.float32).at[:N].set(x)

    deg0, deg1 = _deg_call(dst_p)
    hp, dinv = _tc1_call(
        deg0.reshape(GRID, 1, BLK), deg1.reshape(GRID, 1, BLK), xp, W1)
    s1a, s1b = _agg_call(hp, src_p, dst_p)
    hp2 = _tc2_call(s1a, s1b, dinv, b1)
    s2a, s2b = _agg_call(hp2, src_p, dst_p)
    lo, hi = _tc3_call(s2a, s2b, dinv, W_cat, b_cat)
    return lo[:N], hi[:N]


@jax.jit
def _post(lo, hi):
    mu = lax.bitcast_convert_type(
        jnp.stack([lo[:, :64], hi[:, :64]], axis=-1), jnp.float64)
    lv = lax.bitcast_convert_type(
        jnp.stack([lo[:, 64:], hi[:, 64:]], axis=-1), jnp.float64)
    return mu, lv


def kernel(x, edge_index, W1, b1, W_mu, b_mu, W_var, b_var):
    with jax.enable_x64(False):
        src = edge_index[0].astype(jnp.int32)
        dst = edge_index[1].astype(jnp.int32)
        W_cat = jnp.concatenate([W_mu, W_var], axis=1)
        b_cat = jnp.concatenate([b_mu, b_var], axis=0)
        lo, hi = _run(x.astype(jnp.float32), src, dst,
                      W1.astype(jnp.float32), b1.astype(jnp.float32),
                      W_cat.astype(jnp.float32), b_cat.astype(jnp.float32))
    return _post(lo, hi)


# speedup vs baseline: 3.0857x; 1.0005x over previous
import functools

import jax
import jax.numpy as jnp
from jax import lax
from jax.experimental import pallas as pl
from jax.experimental.pallas import tpu as pltpu
from jax.experimental.pallas import tpu_sc as plsc

N = 10000
D = 128
NC = 2
NS = 16
NW = NC * NS
NB = 80
BATCH = 128
EPW = NB * BATCH
EP = NW * EPW
NP = 10240
RPT = NP // NS
BLK = 1024
GRID = NP // BLK


def _sc_mesh():
    return plsc.VectorSubcoreMesh(
        core_axis_name="c", subcore_axis_name="s",
        num_cores=NC, num_subcores=NS)


def _deg_body(dst_hbm, out0, out1, dst_v, zbuf, ones_v, acc):
    c = lax.axis_index("c")
    s = lax.axis_index("s")
    wid = c * jnp.int32(NS) + s

    def fill_z(i, carry):
        zbuf[pl.ds(i * jnp.int32(16), 16)] = jnp.zeros((16,), jnp.float32)
        return carry

    lax.fori_loop(jnp.int32(0), jnp.int32(RPT // 16), fill_z, 0)

    def fill_o(i, carry):
        ones_v[pl.ds(i * jnp.int32(16), 16)] = jnp.ones((16,), jnp.float32)
        return carry

    lax.fori_loop(jnp.int32(0), jnp.int32(BATCH // 16), fill_o, 0)

    rows = pl.ds(s * jnp.int32(RPT), RPT)
    pltpu.sync_copy(dst_hbm.at[pl.ds(wid * jnp.int32(NB), NB)], dst_v)
    pltpu.sync_copy(zbuf, acc.at[rows])
    plsc.subcore_barrier()

    def body(j, carry):
        pltpu.sync_copy(ones_v, acc.at[dst_v.at[j]], add=True)
        return carry

    lax.fori_loop(jnp.int32(0), jnp.int32(NB), body, 0)
    plsc.subcore_barrier()

    @pl.when(c == 0)
    def _():
        pltpu.sync_copy(acc.at[rows], out0.at[rows])

    @pl.when(c == 1)
    def _():
        pltpu.sync_copy(acc.at[rows], out1.at[rows])


_deg_call = functools.partial(
    pl.kernel,
    out_type=(
        jax.ShapeDtypeStruct((NP,), jnp.float32),
        jax.ShapeDtypeStruct((NP,), jnp.float32),
    ),
    mesh=_sc_mesh(),
    scratch_types=[
        pltpu.VMEM((NB, BATCH), jnp.int32),
        pltpu.VMEM((RPT,), jnp.float32),
        pltpu.VMEM((BATCH,), jnp.float32),
        pltpu.VMEM_SHARED((NP,), jnp.float32),
    ],
)(_deg_body)


CH = 16
TB = EP // BATCH
NBPT = TB // NW


def _emit_chunks(h_hbm, src_hbm, dst_hbm, acc, src_v, dst_v, buf,
                 gsem, isem, base, nchunk):
    i32 = jnp.int32

    def chunk_body(k, carry):
        pk = lax.rem(k, i32(2))
        pn = lax.rem(k + i32(1), i32(2))
        more = k + i32(1) < i32(nchunk)

        @pl.when(more)
        def _():
            pltpu.async_copy(
                src_hbm.at[pl.ds(base + (k + i32(1)) * i32(CH), CH)],
                src_v.at[pn], isem)
            pltpu.async_copy(
                dst_hbm.at[pl.ds(base + (k + i32(1)) * i32(CH), CH)],
                dst_v.at[pn], isem)

        def inner(jj, icarry):
            p = lax.rem(jj, i32(2))
            pnx = lax.rem(jj + i32(1), i32(2))

            @pl.when(jj < i32(CH - 1))
            def _():
                pltpu.async_copy(h_hbm.at[src_v.at[pk, jj + i32(1)]],
                                 buf.at[pnx], gsem.at[pnx])

            pltpu.make_async_copy(h_hbm.at[src_v.at[pk, jj]],
                                  buf.at[p], gsem.at[p]).wait()
            pltpu.sync_copy(buf.at[p], acc.at[dst_v.at[pk, jj]], add=True)
            return icarry

        lax.fori_loop(i32(0), i32(CH), inner, 0)

        @pl.when(more)
        def _():
            pltpu.make_async_copy(
                src_hbm.at[pl.ds(base + (k + i32(1)) * i32(CH), CH)],
                src_v.at[pn], isem).wait()
            pltpu.make_async_copy(
                dst_hbm.at[pl.ds(base + (k + i32(1)) * i32(CH), CH)],
                dst_v.at[pn], isem).wait()
            pltpu.async_copy(h_hbm.at[src_v.at[pn, i32(0)]],
                             buf.at[i32(0)], gsem.at[i32(0)])

        return carry

    lax.fori_loop(i32(0), i32(nchunk), chunk_body, 0)


def _agg_body(h_hbm, src_hbm, dst_hbm, out0, out1,
              src_v, dst_v, buf, acc, gsem, isem):
    c = lax.axis_index("c")
    s = lax.axis_index("s")
    i32 = jnp.int32

    base = (c * i32(NS) + s) * i32(NBPT)
    rows = pl.ds(s * i32(RPT), RPT)

    @pl.when(c == 0)
    def _():
        pltpu.sync_copy(h_hbm.at[rows], acc.at[rows])

    @pl.when(c == 1)
    def _():
        def fill_z(i, carry):
            for k in range(D // 16):
                buf[i32(0), i, pl.ds(i32(16 * k), 16)] = jnp.zeros(
                    (16,), jnp.float32)
            return carry

        lax.fori_loop(i32(0), i32(BATCH), fill_z, 0)
        for r in range(RPT // BATCH):
            pltpu.sync_copy(
                buf.at[i32(0)],
                acc.at[pl.ds(s * i32(RPT) + i32(BATCH * r), BATCH)])

    pltpu.sync_copy(src_hbm.at[pl.ds(base, CH)], src_v.at[i32(0)])
    pltpu.sync_copy(dst_hbm.at[pl.ds(base, CH)], dst_v.at[i32(0)])
    pltpu.async_copy(h_hbm.at[src_v.at[i32(0), i32(0)]],
                     buf.at[i32(0)], gsem.at[i32(0)])
    plsc.subcore_barrier()
    _emit_chunks(h_hbm, src_hbm, dst_hbm, acc, src_v, dst_v, buf,
                 gsem, isem, base, NBPT // CH)
    plsc.subcore_barrier()

    @pl.when(c == 0)
    def _():
        pltpu.sync_copy(acc.at[rows], out0.at[rows])

    @pl.when(c == 1)
    def _():
        pltpu.sync_copy(acc.at[rows], out1.at[rows])


_agg_call = functools.partial(
    pl.kernel,
    out_type=(
        jax.ShapeDtypeStruct((NP, D), jnp.float32),
        jax.ShapeDtypeStruct((NP, D), jnp.float32),
    ),
    mesh=_sc_mesh(),
    scratch_types=[
        pltpu.VMEM((2, CH, BATCH), jnp.int32),
        pltpu.VMEM((2, CH, BATCH), jnp.int32),
        pltpu.VMEM((2, BATCH, D), jnp.float32),
        pltpu.VMEM_SHARED((NP, D), jnp.float32),
        pltpu.SemaphoreType.DMA((2,)),
        pltpu.SemaphoreType.DMA,
    ],
)(_agg_body)


def _tc1_body(deg0_ref, deg1_ref, x_ref, w_ref, h_ref, dinv_ref):
    d = deg0_ref[0, 0, :] + deg1_ref[0, 0, :] + 1.0
    di = lax.rsqrt(d)
    h = jnp.dot(x_ref[...], w_ref[...], preferred_element_type=jnp.float32,
                precision=lax.Precision.HIGHEST)
    h_ref[...] = di[:, None] * h
    dinv_ref[0, 0, :] = di


def _tc2_body(s0_ref, s1_ref, dinv_ref, b_ref, out_ref):
    di = dinv_ref[0, 0, :][:, None]
    s = s0_ref[...] + s1_ref[...]
    h = jnp.maximum(di * s + b_ref[...][None, :], 0.0)
    out_ref[...] = di * h


def _tc3_body(s0_ref, s1_ref, dinv_ref, w_ref, b_ref, lo_ref, hi_ref):
    di = dinv_ref[0, 0, :][:, None]
    a = di * (s0_ref[...] + s1_ref[...])
    y = (
        jnp.dot(a, w_ref[...], preferred_element_type=jnp.float32,
                precision=lax.Precision.HIGHEST)
        + b_ref[...][None, :]
    )
    bits = lax.bitcast_convert_type(y, jnp.uint32)
    sign = bits & jnp.uint32(0x80000000)
    exp8 = (bits >> jnp.uint32(23)) & jnp.uint32(0xFF)
    mant = bits & jnp.uint32(0x7FFFFF)
    exp11 = jnp.where(exp8 == jnp.uint32(255), jnp.uint32(2047),
                      exp8 + jnp.uint32(896))
    hi = sign | (exp11 << jnp.uint32(20)) | (mant >> jnp.uint32(3))
    zero = exp8 == jnp.uint32(0)
    hi_ref[...] = jnp.where(zero, sign, hi)
    lo_ref[...] = jnp.where(zero, jnp.uint32(0), mant << jnp.uint32(29))


def _row_spec(width):
    return pl.BlockSpec((BLK, width), lambda i: (i, 0))


def _vec_spec():
    return pl.BlockSpec((1, 1, BLK), lambda i: (i, 0, 0))


def _full_spec(r, c):
    return pl.BlockSpec((r, c), lambda i: (0, 0))


_tc1_call = pl.pallas_call(
    _tc1_body,
    grid=(GRID,),
    in_specs=[_vec_spec(), _vec_spec(), _row_spec(D), _full_spec(D, D)],
    out_specs=[_row_spec(D), _vec_spec()],
    out_shape=[
        jax.ShapeDtypeStruct((NP, D), jnp.float32),
        jax.ShapeDtypeStruct((GRID, 1, BLK), jnp.float32),
    ],
)

_tc2_call = pl.pallas_call(
    _tc2_body,
    grid=(GRID,),
    in_specs=[
        _row_spec(D),
        _row_spec(D),
        _vec_spec(),
        pl.BlockSpec((D,), lambda i: (0,)),
    ],
    out_specs=_row_spec(D),
    out_shape=jax.ShapeDtypeStruct((NP, D), jnp.float32),
)

_tc3_call = pl.pallas_call(
    _tc3_body,
    grid=(GRID,),
    in_specs=[
        _row_spec(D),
        _row_spec(D),
        _vec_spec(),
        _full_spec(D, D),
        pl.BlockSpec((D,), lambda i: (0,)),
    ],
    out_specs=[_row_spec(D), _row_spec(D)],
    out_shape=[
        jax.ShapeDtypeStruct((NP, D), jnp.uint32),
        jax.ShapeDtypeStruct((NP, D), jnp.uint32),
    ],
)


@jax.jit
def _full(x, edge_index, W1, b1, W_mu, b_mu, W_var, b_var):
    with jax.enable_x64(False):
        src = edge_index[0].astype(jnp.int32)
        dst = edge_index[1].astype(jnp.int32)
        W_cat = jnp.concatenate([W_mu, W_var], axis=1).astype(jnp.float32)
        b_cat = jnp.concatenate([b_mu, b_var], axis=0).astype(jnp.float32)
        x32 = x.astype(jnp.float32)
        W1c = W1.astype(jnp.float32)
        b1c = b1.astype(jnp.float32)

        pad = EP - src.shape[0]
        src_p = jnp.concatenate(
            [src, jnp.arange(pad, dtype=jnp.int32) % N]).reshape(TB, BATCH)
        dst_p = jnp.concatenate(
            [dst, N + jnp.arange(pad, dtype=jnp.int32) % (NP - N)],
        ).reshape(TB, BATCH)
        xp = jnp.zeros((NP, D), jnp.float32).at[:N].set(x32)

        deg0, deg1 = _deg_call(dst_p)
        hp, dinv = _tc1_call(
            deg0.reshape(GRID, 1, BLK), deg1.reshape(GRID, 1, BLK), xp, W1c)
        s1a, s1b = _agg_call(hp, src_p, dst_p)
        hp2 = _tc2_call(s1a, s1b, dinv, b1c)
        s2a, s2b = _agg_call(hp2, src_p, dst_p)
        lo, hi = _tc3_call(s2a, s2b, dinv, W_cat, b_cat)

    mu = lax.bitcast_convert_type(
        jnp.stack([lo[:N, :64], hi[:N, :64]], axis=-1), jnp.float64)
    lv = lax.bitcast_convert_type(
        jnp.stack([lo[:N, 64:], hi[:N, 64:]], axis=-1), jnp.float64)
    return mu, lv


def kernel(x, edge_index, W1, b1, W_mu, b_mu, W_var, b_var):
    return _full(x, edge_index, W1, b1, W_mu, b_mu, W_var, b_var)


# speedup vs baseline: 3.2903x; 1.0663x over previous
import functools

import jax
import jax.numpy as jnp
from jax import lax
from jax.experimental import pallas as pl
from jax.experimental.pallas import tpu as pltpu
from jax.experimental.pallas import tpu_sc as plsc

N = 10000
D = 128
DL = 64
NC = 2
NS = 16
NW = NC * NS
NB = 80
BATCH = 128
EPW = NB * BATCH
EP = NW * EPW
NP = 10240
RPT = NP // NS
BLK = 1024
GRID = NP // BLK


def _sc_mesh():
    return plsc.VectorSubcoreMesh(
        core_axis_name="c", subcore_axis_name="s",
        num_cores=NC, num_subcores=NS)


def _deg_body(dst_hbm, out0, out1, dst_v, zbuf, ones_v, acc):
    c = lax.axis_index("c")
    s = lax.axis_index("s")
    wid = c * jnp.int32(NS) + s

    def fill_z(i, carry):
        zbuf[pl.ds(i * jnp.int32(16), 16)] = jnp.zeros((16,), jnp.float32)
        return carry

    lax.fori_loop(jnp.int32(0), jnp.int32(RPT // 16), fill_z, 0)

    def fill_o(i, carry):
        ones_v[pl.ds(i * jnp.int32(16), 16)] = jnp.ones((16,), jnp.float32)
        return carry

    lax.fori_loop(jnp.int32(0), jnp.int32(BATCH // 16), fill_o, 0)

    rows = pl.ds(s * jnp.int32(RPT), RPT)
    pltpu.sync_copy(dst_hbm.at[pl.ds(wid * jnp.int32(NB), NB)], dst_v)
    pltpu.sync_copy(zbuf, acc.at[rows])
    plsc.subcore_barrier()

    def body(j, carry):
        pltpu.sync_copy(ones_v, acc.at[dst_v.at[j]], add=True)
        return carry

    lax.fori_loop(jnp.int32(0), jnp.int32(NB), body, 0)
    plsc.subcore_barrier()

    @pl.when(c == 0)
    def _():
        pltpu.sync_copy(acc.at[rows], out0.at[rows])

    @pl.when(c == 1)
    def _():
        pltpu.sync_copy(acc.at[rows], out1.at[rows])


_deg_call = functools.partial(
    pl.kernel,
    out_type=(
        jax.ShapeDtypeStruct((NP,), jnp.float32),
        jax.ShapeDtypeStruct((NP,), jnp.float32),
    ),
    mesh=_sc_mesh(),
    scratch_types=[
        pltpu.VMEM((NB, BATCH), jnp.int32),
        pltpu.VMEM((RPT,), jnp.float32),
        pltpu.VMEM((BATCH,), jnp.float32),
        pltpu.VMEM_SHARED((NP,), jnp.float32),
    ],
)(_deg_body)


CH = 16
TB = EP // BATCH
NBPT = TB // NW


def _emit_chunks(h_hbm, src_hbm, dst_hbm, acc, src_v, dst_v, buf,
                 gsem, isem, base, nchunk):
    i32 = jnp.int32

    def chunk_body(k, carry):
        pk = lax.rem(k, i32(2))
        pn = lax.rem(k + i32(1), i32(2))
        more = k + i32(1) < i32(nchunk)

        @pl.when(more)
        def _():
            pltpu.async_copy(
                src_hbm.at[pl.ds(base + (k + i32(1)) * i32(CH), CH)],
                src_v.at[pn], isem)
            pltpu.async_copy(
                dst_hbm.at[pl.ds(base + (k + i32(1)) * i32(CH), CH)],
                dst_v.at[pn], isem)

        def inner(jj, icarry):
            p = lax.rem(jj, i32(2))
            pnx = lax.rem(jj + i32(1), i32(2))

            @pl.when(jj < i32(CH - 1))
            def _():
                pltpu.async_copy(h_hbm.at[src_v.at[pk, jj + i32(1)]],
                                 buf.at[pnx], gsem.at[pnx])

            pltpu.make_async_copy(h_hbm.at[src_v.at[pk, jj]],
                                  buf.at[p], gsem.at[p]).wait()
            pltpu.sync_copy(buf.at[p], acc.at[dst_v.at[pk, jj]], add=True)
            return icarry

        lax.fori_loop(i32(0), i32(CH), inner, 0)

        @pl.when(more)
        def _():
            pltpu.make_async_copy(
                src_hbm.at[pl.ds(base + (k + i32(1)) * i32(CH), CH)],
                src_v.at[pn], isem).wait()
            pltpu.make_async_copy(
                dst_hbm.at[pl.ds(base + (k + i32(1)) * i32(CH), CH)],
                dst_v.at[pn], isem).wait()
            pltpu.async_copy(h_hbm.at[src_v.at[pn, i32(0)]],
                             buf.at[i32(0)], gsem.at[i32(0)])

        return carry

    lax.fori_loop(i32(0), i32(nchunk), chunk_body, 0)


def _agg_body(h_hbm, src_hbm, dst_hbm, out0, out1,
              src_v, dst_v, buf, acc, gsem, isem):
    c = lax.axis_index("c")
    s = lax.axis_index("s")
    i32 = jnp.int32

    base = (c * i32(NS) + s) * i32(NBPT)
    rows = pl.ds(s * i32(RPT), RPT)

    @pl.when(c == 0)
    def _():
        pltpu.sync_copy(h_hbm.at[rows], acc.at[rows])

    @pl.when(c == 1)
    def _():
        def fill_z(i, carry):
            for k in range(D // 16):
                buf[i32(0), i, pl.ds(i32(16 * k), 16)] = jnp.zeros(
                    (16,), jnp.float32)
            return carry

        lax.fori_loop(i32(0), i32(BATCH), fill_z, 0)
        for r in range(RPT // BATCH):
            pltpu.sync_copy(
                buf.at[i32(0)],
                acc.at[pl.ds(s * i32(RPT) + i32(BATCH * r), BATCH)])

    pltpu.sync_copy(src_hbm.at[pl.ds(base, CH)], src_v.at[i32(0)])
    pltpu.sync_copy(dst_hbm.at[pl.ds(base, CH)], dst_v.at[i32(0)])
    pltpu.async_copy(h_hbm.at[src_v.at[i32(0), i32(0)]],
                     buf.at[i32(0)], gsem.at[i32(0)])
    plsc.subcore_barrier()
    _emit_chunks(h_hbm, src_hbm, dst_hbm, acc, src_v, dst_v, buf,
                 gsem, isem, base, NBPT // CH)
    plsc.subcore_barrier()

    @pl.when(c == 0)
    def _():
        pltpu.sync_copy(acc.at[rows], out0.at[rows])

    @pl.when(c == 1)
    def _():
        pltpu.sync_copy(acc.at[rows], out1.at[rows])


_agg_call = functools.partial(
    pl.kernel,
    out_type=(
        jax.ShapeDtypeStruct((NP, D), jnp.float32),
        jax.ShapeDtypeStruct((NP, D), jnp.float32),
    ),
    mesh=_sc_mesh(),
    scratch_types=[
        pltpu.VMEM((2, CH, BATCH), jnp.int32),
        pltpu.VMEM((2, CH, BATCH), jnp.int32),
        pltpu.VMEM((2, BATCH, D), jnp.float32),
        pltpu.VMEM_SHARED((NP, D), jnp.float32),
        pltpu.SemaphoreType.DMA((2,)),
        pltpu.SemaphoreType.DMA,
    ],
)(_agg_body)


def _tc1_body(deg0_ref, deg1_ref, x_ref, w_ref, h_ref, dinv_ref):
    d = deg0_ref[0, 0, :] + deg1_ref[0, 0, :] + 1.0
    di = lax.rsqrt(d)
    h = jnp.dot(x_ref[...], w_ref[...], preferred_element_type=jnp.float32,
                precision=lax.Precision.HIGHEST)
    h_ref[...] = di[:, None] * h
    dinv_ref[0, 0, :] = di


def _tc2_body(s0_ref, s1_ref, dinv_ref, b_ref, out_ref):
    di = dinv_ref[0, 0, :][:, None]
    s = s0_ref[...] + s1_ref[...]
    h = jnp.maximum(di * s + b_ref[...][None, :], 0.0)
    out_ref[...] = di * h


def _tc3_body(s0_ref, s1_ref, dinv_ref, w_ref, b_ref, mu_ref, lv_ref):
    di = dinv_ref[0, 0, :][:, None]
    a = di * (s0_ref[...] + s1_ref[...])
    y = (
        jnp.dot(a, w_ref[...], preferred_element_type=jnp.float32,
                precision=lax.Precision.HIGHEST)
        + b_ref[...][None, :]
    )
    mu_ref[...] = y[:, :DL]
    lv_ref[...] = y[:, DL:]


def _row_spec(width):
    return pl.BlockSpec((BLK, width), lambda i: (i, 0))


def _vec_spec():
    return pl.BlockSpec((1, 1, BLK), lambda i: (i, 0, 0))


def _full_spec(r, c):
    return pl.BlockSpec((r, c), lambda i: (0, 0))


_tc1_call = pl.pallas_call(
    _tc1_body,
    grid=(GRID,),
    in_specs=[_vec_spec(), _vec_spec(), _row_spec(D), _full_spec(D, D)],
    out_specs=[_row_spec(D), _vec_spec()],
    out_shape=[
        jax.ShapeDtypeStruct((NP, D), jnp.float32),
        jax.ShapeDtypeStruct((GRID, 1, BLK), jnp.float32),
    ],
)

_tc2_call = pl.pallas_call(
    _tc2_body,
    grid=(GRID,),
    in_specs=[
        _row_spec(D),
        _row_spec(D),
        _vec_spec(),
        pl.BlockSpec((D,), lambda i: (0,)),
    ],
    out_specs=_row_spec(D),
    out_shape=jax.ShapeDtypeStruct((NP, D), jnp.float32),
)

_tc3_call = pl.pallas_call(
    _tc3_body,
    grid=(GRID,),
    in_specs=[
        _row_spec(D),
        _row_spec(D),
        _vec_spec(),
        _full_spec(D, D),
        pl.BlockSpec((D,), lambda i: (0,)),
    ],
    out_specs=[_row_spec(DL), _row_spec(DL)],
    out_shape=[
        jax.ShapeDtypeStruct((NP, DL), jnp.float32),
        jax.ShapeDtypeStruct((NP, DL), jnp.float32),
    ],
)


@jax.jit
def _full(x, edge_index, W1, b1, W_mu, b_mu, W_var, b_var):
    with jax.enable_x64(False):
        src = edge_index[0].astype(jnp.int32)
        dst = edge_index[1].astype(jnp.int32)
        W_cat = jnp.concatenate([W_mu, W_var], axis=1).astype(jnp.float32)
        b_cat = jnp.concatenate([b_mu, b_var], axis=0).astype(jnp.float32)
        x32 = x.astype(jnp.float32)
        W1c = W1.astype(jnp.float32)
        b1c = b1.astype(jnp.float32)

        pad = EP - src.shape[0]
        src_p = jnp.concatenate(
            [src, jnp.arange(pad, dtype=jnp.int32) % N]).reshape(TB, BATCH)
        dst_p = jnp.concatenate(
            [dst, N + jnp.arange(pad, dtype=jnp.int32) % (NP - N)],
        ).reshape(TB, BATCH)
        xp = jnp.zeros((NP, D), jnp.float32).at[:N].set(x32)

        deg0, deg1 = _deg_call(dst_p)
        hp, dinv = _tc1_call(
            deg0.reshape(GRID, 1, BLK), deg1.reshape(GRID, 1, BLK), xp, W1c)
        s1a, s1b = _agg_call(hp, src_p, dst_p)
        hp2 = _tc2_call(s1a, s1b, dinv, b1c)
        s2a, s2b = _agg_call(hp2, src_p, dst_p)
        mu32, lv32 = _tc3_call(s2a, s2b, dinv, W_cat, b_cat)

    return mu32[:N].astype(jnp.float64), lv32[:N].astype(jnp.float64)


def kernel(x, edge_index, W1, b1, W_mu, b_mu, W_var, b_var):
    return _full(x, edge_index, W1, b1, W_mu, b_mu, W_var, b_var)
